# baseline (speedup 1.0000x reference)
import functools

import jax
import jax.numpy as jnp
import numpy as np
from jax import lax
from jax.experimental import pallas as pl
from jax.experimental.pallas import tpu as pltpu
from jax.experimental.pallas import tpu_sc as plsc

_LANES = 16
_CHUNK_ROWS = 128
_ROWS_PER_BLOCK = 8192
_SC_ROWS = 40960


def _half_argmax(xh, offset):
    m = jnp.max(xh, axis=1, keepdims=True)
    col = lax.broadcasted_iota(jnp.int32, xh.shape, 1)
    cand = jnp.where(xh == m, col + offset, 2 * xh.shape[1])
    return m[:, 0], jnp.min(cand, axis=1)


def _argmax_body(xl_ref, xr_ref, idx_ref):
    half = xl_ref.shape[1]
    ml, il = _half_argmax(xl_ref[...], 0)
    mr, ir = _half_argmax(xr_ref[...], half)
    idx_ref[...] = jnp.where(mr > ml, ir, il)


def _tc_argmax(x, n_rows):
    _, c = x.shape
    r = min(_ROWS_PER_BLOCK, n_rows)
    h = c // 2
    return pl.pallas_call(
        _argmax_body,
        grid=(n_rows // r,),
        in_specs=[
            pl.BlockSpec((r, h), lambda i: (i, 0)),
            pl.BlockSpec((r, h), lambda i: (i, 1)),
        ],
        out_specs=pl.BlockSpec((r,), lambda i: (i,)),
        out_shape=jax.ShapeDtypeStruct((n_rows,), jnp.int32),
    )(x, x)


def _sc_finalize(mapping, idx, sc_part, n_total):
    n_tc = idx.shape[0]
    n_sc = sc_part.shape[0] // 2
    info = plsc.get_sparse_core_info()
    nw = info.num_cores * info.num_subcores
    bpw = n_tc // nw
    tpw = n_sc // nw
    mesh = plsc.VectorSubcoreMesh(core_axis_name="c", subcore_axis_name="s")

    @functools.partial(
        pl.kernel,
        mesh=mesh,
        out_type=jax.ShapeDtypeStruct((2 * n_total,), jnp.float32),
        scratch_types=[
            pltpu.VMEM((2 * mapping.shape[0],), jnp.float32),
            pltpu.VMEM((bpw,), jnp.int32),
            pltpu.VMEM((2 * bpw,), jnp.float32),
            pltpu.VMEM((2 * tpw,), jnp.float32),
        ],
        compiler_params=pltpu.CompilerParams(needs_layout_passes=False),
    )
    def _k(map_hbm, idx_hbm, sc_hbm, out_hbm, tbl_v, idx_v, out_v, cp_v):
        wid = lax.axis_index("s") * info.num_cores + lax.axis_index("c")
        base = wid * bpw
        pltpu.sync_copy(map_hbm, tbl_v)
        pltpu.sync_copy(idx_hbm.at[pl.ds(base, bpw)], idx_v)
        lane = lax.iota(jnp.int32, _LANES)
        zeros = lane * 0
        ones = zeros + 1

        def body(i, carry):
            off = pl.multiple_of(i * _LANES, _LANES)
            iv = idx_v[pl.ds(off, _LANES)]
            first = plsc.load_gather(tbl_v, [iv * 2])
            second = plsc.load_gather(tbl_v, [iv * 2 + 1])
            pos = (lane + off) * 2
            plsc.store_scatter(out_v, [pos], first)
            plsc.store_scatter(out_v, [pos + 1], second)
            return carry

        lax.fori_loop(0, bpw // _LANES, body, 0)
        pltpu.sync_copy(out_v, out_hbm.at[pl.ds(2 * base, 2 * bpw)])
        pltpu.sync_copy(sc_hbm.at[pl.ds(2 * wid * tpw, 2 * tpw)], cp_v)
        pltpu.sync_copy(
            cp_v, out_hbm.at[pl.ds(2 * (n_tc + wid * tpw), 2 * tpw)])

    return _k(mapping.reshape(-1), idx, sc_part)


def _sc_qam_encode(x, mapping, n_rows, row_offset):
    _, n_cols = x.shape
    info = plsc.get_sparse_core_info()
    nw = info.num_cores * info.num_subcores
    rpw = n_rows // nw
    ch = _CHUNK_ROWS
    n_chunks = rpw // ch
    groups = ch // _LANES
    mesh = plsc.VectorSubcoreMesh(core_axis_name="c", subcore_axis_name="s")
    neg_inf = np.float32(-np.inf)

    @functools.partial(
        pl.kernel,
        mesh=mesh,
        out_type=jax.ShapeDtypeStruct((2 * n_rows,), jnp.float32),
        scratch_types=[
            pltpu.VMEM((ch, n_cols), jnp.float32),
            pltpu.VMEM((ch, n_cols), jnp.float32),
            pltpu.VMEM((2 * mapping.shape[0],), jnp.float32),
            pltpu.VMEM((2 * rpw,), jnp.float32),
            pltpu.SemaphoreType.DMA,
            pltpu.SemaphoreType.DMA,
        ],
        compiler_params=pltpu.CompilerParams(needs_layout_passes=False),
    )
    def _k(x_hbm, map_hbm, out_hbm, buf0, buf1, tbl_v, out_v, sem0, sem1):
        wid = lax.axis_index("s") * info.num_cores + lax.axis_index("c")
        row0 = wid * rpw
        pltpu.sync_copy(map_hbm, tbl_v)

        def chunk_src(k):
            return x_hbm.at[pl.ds(row_offset + row0 + k * ch, ch), :]

        lane = lax.iota(jnp.int32, _LANES)
        minf = lane.astype(jnp.float32) * 0.0 + neg_inf
        zeros = lane * 0
        ones = zeros + 1

        def process(buf, k):
            def group(g, carry):
                rowv = lane + g * _LANES
                colv = lane
                m_a = plsc.load_gather(buf, [rowv, colv])
                ca = colv
                for t in range(1, n_cols - _LANES + 1):
                    colv = lane + t
                    v = plsc.load_gather(buf, [rowv, colv])
                    gt = v > m_a
                    m_a = jnp.where(gt, v, m_a)
                    ca = jnp.where(gt, colv, ca)
                m_b = minf
                cb = zeros
                for t in range(n_cols - _LANES + 1, n_cols):
                    w = lane >= (n_cols - t)
                    colv = lane + t - w.astype(jnp.int32) * n_cols
                    v = plsc.load_gather(buf, [rowv, colv])
                    gta = (v > m_a) & (~w)
                    gtb = (v > m_b) & w
                    m_a = jnp.where(gta, v, m_a)
                    ca = jnp.where(gta, colv, ca)
                    m_b = jnp.where(gtb, v, m_b)
                    cb = jnp.where(gtb, colv, cb)
                best = jnp.where(m_b >= m_a, cb, ca)
                first = plsc.load_gather(tbl_v, [best * 2])
                second = plsc.load_gather(tbl_v, [best * 2 + 1])
                pos = (rowv + k * ch) * 2
                plsc.store_scatter(out_v, [pos], first)
                plsc.store_scatter(out_v, [pos + 1], second)
                return carry

            lax.fori_loop(0, groups, group, 0)

        pltpu.async_copy(chunk_src(0), buf0, sem0).wait()

        def super_step(i, carry):
            k0 = i * 2

            @pl.when(k0 + 1 < n_chunks)
            def _():
                pltpu.async_copy(chunk_src(k0 + 1), buf1, sem1)

            process(buf0, k0)

            @pl.when(k0 + 2 < n_chunks)
            def _():
                pltpu.async_copy(chunk_src(k0 + 2), buf0, sem0)

            @pl.when(k0 + 1 < n_chunks)
            def _():
                pltpu.make_async_copy(chunk_src(k0 + 1), buf1, sem1).wait()
                process(buf1, k0 + 1)

            @pl.when(k0 + 2 < n_chunks)
            def _():
                pltpu.make_async_copy(chunk_src(k0 + 2), buf0, sem0).wait()

            return carry

        lax.fori_loop(0, (n_chunks + 1) // 2, super_step, 0)
        pltpu.sync_copy(out_v, out_hbm.at[pl.ds(2 * row0, 2 * rpw)])

    return _k(x, mapping.reshape(-1))


def kernel(x, mapping):
    n, _ = x.shape
    n_tc = n - _SC_ROWS
    sc_flat = _sc_qam_encode(x, mapping, _SC_ROWS, n_tc)
    idx = _tc_argmax(x, n_tc)
    return _sc_finalize(mapping, idx, sc_flat, n).reshape(n, 2)


# speedup vs baseline: 1.2009x; 1.2009x over previous
import functools

import jax
import jax.numpy as jnp
from jax import lax
from jax.experimental import pallas as pl
from jax.experimental.pallas import tpu as pltpu
from jax.experimental.pallas import tpu_sc as plsc

_ROWS_PER_BLOCK = 8192
_LANES = 16


def _argmax_body(x_ref, idx_ref):
    xb = x_ref[...]
    m = jnp.max(xb, axis=1, keepdims=True)
    col = lax.broadcasted_iota(jnp.int32, xb.shape, 1)
    cand = jnp.where(xb == m, col, xb.shape[1])
    idx_ref[...] = jnp.min(cand, axis=1)


def _tc_argmax(x):
    n, c = x.shape
    r = _ROWS_PER_BLOCK
    return pl.pallas_call(
        _argmax_body,
        grid=(n // r,),
        in_specs=[pl.BlockSpec((r, c), lambda i: (i, 0))],
        out_specs=pl.BlockSpec((r,), lambda i: (i,)),
        out_shape=jax.ShapeDtypeStruct((n,), jnp.int32),
    )(x)


def _sc_lookup(table_flat, idx):
    n = idx.shape[0]
    info = plsc.get_sparse_core_info()
    nw = info.num_cores * info.num_subcores
    bpw = n // nw
    mesh = plsc.VectorSubcoreMesh(core_axis_name="c", subcore_axis_name="s")

    @functools.partial(
        pl.kernel,
        mesh=mesh,
        out_type=jax.ShapeDtypeStruct((2 * n,), jnp.float32),
        scratch_types=[
            pltpu.VMEM((table_flat.shape[0],), jnp.float32),
            pltpu.VMEM((bpw,), jnp.int32),
            pltpu.VMEM((2 * bpw,), jnp.float32),
        ],
        compiler_params=pltpu.CompilerParams(needs_layout_passes=False),
    )
    def _k(table_hbm, idx_hbm, out_hbm, tbl_v, idx_v, out_v):
        wid = lax.axis_index("s") * info.num_cores + lax.axis_index("c")
        base = wid * bpw
        pltpu.sync_copy(table_hbm, tbl_v)
        pltpu.sync_copy(idx_hbm.at[pl.ds(base, bpw)], idx_v)

        def body(i, carry):
            off = pl.multiple_of(i * _LANES, _LANES)
            iv = idx_v[pl.ds(off, _LANES)]
            first = plsc.load_gather(tbl_v, [iv * 2])
            second = plsc.load_gather(tbl_v, [iv * 2 + 1])
            pos = (lax.iota(jnp.int32, _LANES) + off) * 2
            plsc.store_scatter(out_v, [pos], first)
            plsc.store_scatter(out_v, [pos + 1], second)
            return carry

        lax.fori_loop(0, bpw // _LANES, body, 0)
        pltpu.sync_copy(out_v, out_hbm.at[pl.ds(2 * base, 2 * bpw)])

    return _k(table_flat, idx)


def kernel(x, mapping):
    idx = _tc_argmax(x)
    flat = _sc_lookup(mapping.reshape(-1), idx)
    return flat.reshape(x.shape[0], 2)
